# Initial kernel scaffold; baseline (speedup 1.0000x reference)
import functools

import jax
import jax.numpy as jnp
from jax import lax
from jax.experimental import pallas as pl
from jax.experimental.pallas import tpu as pltpu
from jax.experimental.pallas import tpu_sc as plsc

H = 128
C = 80
NC = 2
NS = 16


def _zeros16f():
    return jnp.zeros((16,), jnp.float32)


def _full16(v):
    return jnp.full((16,), v, jnp.int32)


def _scale_chunk(rb, normb, c):
    def body(e, carry):
        nv = plsc.load_gather(normb, [_full16(c), _full16(e)])
        for j in range(H // 16):
            sl = pl.ds(j * 16, 16)
            rb[e, sl] = rb[e, sl] * nv
        return carry
    lax.fori_loop(0, C, body, 0)


def _aggregate(h_hbm, srcb, dstb, normb, acc, rows_per_tile, rb0, gsem, ssem):
    def body(c, carry):
        pltpu.async_copy(h_hbm.at[srcb.at[c]], rb0, gsem).wait()
        _scale_chunk(rb0, normb, c)
        pltpu.async_copy(rb0, acc.at[dstb.at[c]], ssem, add=True).wait()
        return carry
    lax.fori_loop(0, rows_per_tile, body, 0)


def _zero_acc_slice(rb0, acc, base, npt):
    def zbody(e, carry):
        for j in range(H // 16):
            rb0[e, pl.ds(j * 16, 16)] = _zeros16f()
        return carry
    lax.fori_loop(0, C, zbody, 0)
    for m in range(npt // C):
        pltpu.sync_copy(rb0, acc.at[pl.ds(base + m * C, C)])


def _sc_layer1(h1, src2d, dst2d, ew2d, NP):
    ROWS = src2d.shape[0]
    RPT = ROWS // (NC * NS)
    RPS = ROWS // NS
    NPT = NP // NS
    mesh = plsc.VectorSubcoreMesh(core_axis_name="c", subcore_axis_name="s")

    @functools.partial(
        pl.kernel,
        out_type=[
            jax.ShapeDtypeStruct((NC, NP, H), jnp.float32),
            jax.ShapeDtypeStruct((ROWS, C), jnp.float32),
            jax.ShapeDtypeStruct((NP,), jnp.float32),
        ],
        mesh=mesh,
        scratch_types=dict(
            acc=pltpu.VMEM_SHARED((NP, H), jnp.float32),
            deg_sh=pltpu.VMEM_SHARED((NP,), jnp.float32),
            dinv_sh=pltpu.VMEM_SHARED((NP,), jnp.float32),
            ewb=pltpu.VMEM((RPS, C), jnp.float32),
            dstb2=pltpu.VMEM((RPS, C), jnp.int32),
            degb=pltpu.VMEM((NPT,), jnp.float32),
            dslice=pltpu.VMEM((NPT,), jnp.float32),
            d2slice=pltpu.VMEM((NPT,), jnp.float32),
            dinvb=pltpu.VMEM((NP,), jnp.float32),
            srcb=pltpu.VMEM((RPT, C), jnp.int32),
            dstb=pltpu.VMEM((RPT, C), jnp.int32),
            ewbn=pltpu.VMEM((RPT, C), jnp.float32),
            normb=pltpu.VMEM((RPT, C), jnp.float32),
            rb0=pltpu.VMEM((C, H), jnp.float32),
            gsem=pltpu.SemaphoreType.DMA,
            ssem=pltpu.SemaphoreType.DMA,
        ),
    )
    def k(h1_hbm, src_hbm, dst_hbm, ew_hbm, part_out, norm_out, dinv2_out,
          acc, deg_sh, dinv_sh, ewb, dstb2, degb, dslice, d2slice, dinvb,
          srcb, dstb, ewbn, normb, rb0, gsem, ssem):
        cid = lax.axis_index("c")
        sid = lax.axis_index("s")
        wid = cid * NS + sid
        nbase = sid * NPT

        def zd(kk, carry):
            dinvb[pl.ds(kk * 16, 16)] = _zeros16f()
            return carry
        lax.fori_loop(0, NPT // 16, zd, 0)
        pltpu.sync_copy(dinvb.at[pl.ds(0, NPT)], deg_sh.at[pl.ds(nbase, NPT)])
        _zero_acc_slice(rb0, acc, nbase, NPT)
        plsc.subcore_barrier()

        ebase = sid * RPS
        pltpu.sync_copy(ew_hbm.at[pl.ds(ebase, RPS)], ewb)
        pltpu.sync_copy(dst_hbm.at[pl.ds(ebase, RPS)], dstb2)
        def degbody(c, carry):
            pltpu.sync_copy(ewb.at[c], deg_sh.at[dstb2.at[c]], add=True)
            return carry
        lax.fori_loop(0, RPS, degbody, 0)
        plsc.subcore_barrier()

        pltpu.sync_copy(deg_sh.at[pl.ds(nbase, NPT)], degb)
        def dbody(kk, carry):
            sl = pl.ds(kk * 16, 16)
            dv = degb[sl] + 1.0
            iv = jnp.int32(0x5F3759DF) - (plsc.bitcast(dv, jnp.int32) >> 1)
            y = plsc.bitcast(iv, jnp.float32)
            y = y * (1.5 - 0.5 * dv * y * y)
            y = y * (1.5 - 0.5 * dv * y * y)
            y = y * (1.5 - 0.5 * dv * y * y)
            dslice[sl] = y
            d2slice[sl] = y * y
            return carry
        lax.fori_loop(0, NPT // 16, dbody, 0)
        pltpu.sync_copy(dslice, dinv_sh.at[pl.ds(nbase, NPT)])

        @pl.when(cid == 0)
        def _():
            pltpu.sync_copy(d2slice, dinv2_out.at[pl.ds(nbase, NPT)])
        plsc.subcore_barrier()

        pltpu.sync_copy(dinv_sh, dinvb)
        abase = wid * RPT
        pltpu.sync_copy(src_hbm.at[pl.ds(abase, RPT)], srcb)
        pltpu.sync_copy(dst_hbm.at[pl.ds(abase, RPT)], dstb)
        pltpu.sync_copy(ew_hbm.at[pl.ds(abase, RPT)], ewbn)
        def nbody(c, carry):
            for kk in range(C // 16):
                sl = pl.ds(kk * 16, 16)
                a = plsc.load_gather(dinvb, [srcb[c, sl]])
                b = plsc.load_gather(dinvb, [dstb[c, sl]])
                normb[c, sl] = a * ewbn[c, sl] * b
            return carry
        lax.fori_loop(0, RPT, nbody, 0)
        pltpu.sync_copy(normb, norm_out.at[pl.ds(abase, RPT)])

        _aggregate(h1_hbm, srcb, dstb, normb, acc, RPT, rb0, gsem, ssem)
        plsc.subcore_barrier()

        pltpu.sync_copy(acc.at[pl.ds(nbase, NPT)],
                        part_out.at[cid, pl.ds(nbase, NPT)])

    return k(h1, src2d, dst2d, ew2d)


def _sc_layer2(h2, src2d, dst2d, norm2d, NP):
    ROWS = src2d.shape[0]
    RPT = ROWS // (NC * NS)
    NPT = NP // NS
    mesh = plsc.VectorSubcoreMesh(core_axis_name="c", subcore_axis_name="s")

    @functools.partial(
        pl.kernel,
        out_type=jax.ShapeDtypeStruct((NC, NP, H), jnp.float32),
        mesh=mesh,
        scratch_types=dict(
            acc=pltpu.VMEM_SHARED((NP, H), jnp.float32),
            srcb=pltpu.VMEM((RPT, C), jnp.int32),
            dstb=pltpu.VMEM((RPT, C), jnp.int32),
            normb=pltpu.VMEM((RPT, C), jnp.float32),
            rb0=pltpu.VMEM((C, H), jnp.float32),
            gsem=pltpu.SemaphoreType.DMA,
            ssem=pltpu.SemaphoreType.DMA,
        ),
    )
    def k(h2_hbm, src_hbm, dst_hbm, norm_hbm, part_out,
          acc, srcb, dstb, normb, rb0, gsem, ssem):
        cid = lax.axis_index("c")
        sid = lax.axis_index("s")
        wid = cid * NS + sid
        nbase = sid * NPT
        _zero_acc_slice(rb0, acc, nbase, NPT)
        abase = wid * RPT
        pltpu.sync_copy(src_hbm.at[pl.ds(abase, RPT)], srcb)
        pltpu.sync_copy(dst_hbm.at[pl.ds(abase, RPT)], dstb)
        pltpu.sync_copy(norm_hbm.at[pl.ds(abase, RPT)], normb)
        plsc.subcore_barrier()
        _aggregate(h2_hbm, srcb, dstb, normb, acc, RPT, rb0, gsem, ssem)
        plsc.subcore_barrier()
        pltpu.sync_copy(acc.at[pl.ds(nbase, NPT)],
                        part_out.at[cid, pl.ds(nbase, NPT)])

    return k(h2, src2d, dst2d, norm2d)


_BR = 1280


def _tc_grid_call(body, n_out, NP, *args):
    specs = []
    for a in args:
        if a.ndim == 2 and a.shape[0] == NP:
            specs.append(pl.BlockSpec((_BR, a.shape[1]), lambda i: (i, 0)))
        else:
            specs.append(pl.BlockSpec(a.shape, lambda i: (0,) * a.ndim))
    outs = [jax.ShapeDtypeStruct((NP, H), jnp.float32)] * n_out
    return pl.pallas_call(
        body,
        grid=(NP // _BR,),
        in_specs=specs,
        out_specs=[pl.BlockSpec((_BR, H), lambda i: (i, 0))] * n_out,
        out_shape=outs,
    )(*args)


def _tc1_body(x, wgc1, wm1, bm1, wm2, bm2, h1o, hno):
    xb = x[...]
    h1o[...] = jnp.dot(xb, wgc1[...], preferred_element_type=jnp.float32)
    t = jnp.tanh(jnp.dot(xb, wm1[...], preferred_element_type=jnp.float32)
                 + bm1[...])
    hno[...] = (jnp.dot(t, wm2[...], preferred_element_type=jnp.float32)
                + bm2[...])


def _tc2_body(p0, p1, h1, d2, bgc1, wgc2, h2o):
    agg = p0[...] + p1[...] + d2[...] * h1[...] + bgc1[...]
    g = jnp.maximum(agg, 0.0)
    h2o[...] = jnp.dot(g, wgc2[...], preferred_element_type=jnp.float32)


def _tc3_body(p0, p1, h2, d2, bgc2, hn, wga, wgb, bg, dxo):
    agg = p0[...] + p1[...] + d2[...] * h2[...] + bgc2[...]
    hnb = hn[...]
    z = (jnp.dot(agg, wga[...], preferred_element_type=jnp.float32)
         + jnp.dot(hnb, wgb[...], preferred_element_type=jnp.float32)
         + bg[...])
    gate = jax.nn.sigmoid(z)
    dxo[...] = gate * agg + (1.0 - gate) * hnb


def kernel(t, x, edge_index, edge_weight, W_gc1, b_gc1, W_gc2, b_gc2,
           W_m1, b_m1, W_m2, b_m2, W_g, b_g):
    b_sz, n, h_dim = x.shape
    e_num = edge_weight.shape[0]
    assert h_dim == H and e_num % (C * NC * NS) == 0
    NP = ((n + NS * 16 - 1) // (NS * 16)) * (NS * 16)
    x_flat = x.reshape(n, h_dim)
    xp = jnp.pad(x_flat, ((0, NP - n), (0, 0)))
    src2d = edge_index[0].reshape(-1, C)
    dst2d = edge_index[1].reshape(-1, C)
    ew2d = edge_weight.reshape(-1, C)

    h1, hn = _tc_grid_call(_tc1_body, 2, NP, xp, W_gc1, W_m1,
                           b_m1.reshape(1, H), W_m2, b_m2.reshape(1, H))

    part1, norm2d, dinv2 = _sc_layer1(h1, src2d, dst2d, ew2d, NP)
    d2 = dinv2.reshape(NP, 1)

    (h2,) = _tc_grid_call(_tc2_body, 1, NP, part1[0], part1[1], h1, d2,
                          b_gc1.reshape(1, H), W_gc2)

    part2 = _sc_layer2(h2, src2d, dst2d, norm2d, NP)

    (dx,) = _tc_grid_call(_tc3_body, 1, NP, part2[0], part2[1], h2, d2,
                          b_gc2.reshape(1, H), hn, W_g[:H], W_g[H:],
                          b_g.reshape(1, H))
    return dx[:n].reshape(b_sz, n, h_dim)


# speedup vs baseline: 13.4924x; 13.4924x over previous
import functools

import jax
import jax.numpy as jnp
from jax import lax
from jax.experimental import pallas as pl
from jax.experimental.pallas import tpu as pltpu
from jax.experimental.pallas import tpu_sc as plsc

H = 128
C = 80
NC = 2
NS = 16


def _zeros16f():
    return jnp.zeros((16,), jnp.float32)


def _full16(v):
    return jnp.full((16,), v, jnp.int32)


def _scale_chunk(rb, normb, c):
    def body(e, carry):
        nv = plsc.load_gather(normb, [_full16(c), _full16(e)])
        for j in range(H // 16):
            sl = pl.ds(j * 16, 16)
            rb[e, sl] = rb[e, sl] * nv
        return carry
    lax.fori_loop(0, C, body, 0)


def _aggregate(h_hbm, srcb, dstb, normb, acc, rows_per_tile, rb0, gsem, ssem):
    def body(c, carry):
        pltpu.async_copy(h_hbm.at[srcb.at[c]], rb0, gsem).wait()
        _scale_chunk(rb0, normb, c)
        pltpu.async_copy(rb0, acc.at[dstb.at[c]], ssem, add=True).wait()
        return carry
    lax.fori_loop(0, rows_per_tile, body, 0)


def _zero_acc_slice(rb0, acc, base, npt):
    def zbody(e, carry):
        for j in range(H // 16):
            rb0[e, pl.ds(j * 16, 16)] = _zeros16f()
        return carry
    lax.fori_loop(0, C, zbody, 0)
    for m in range(npt // C):
        pltpu.sync_copy(rb0, acc.at[pl.ds(base + m * C, C)])


def _sc_layer1(h1, src2d, dst2d, ew2d, NP):
    ROWS = src2d.shape[0]
    RPT = ROWS // (NC * NS)
    RPS = ROWS // NS
    NPT = NP // NS
    DB = 50
    AB = 25
    mesh = plsc.VectorSubcoreMesh(core_axis_name="c", subcore_axis_name="s")

    @functools.partial(
        pl.kernel,
        out_type=[
            jax.ShapeDtypeStruct((NC, NP, H), jnp.float32),
            jax.ShapeDtypeStruct((ROWS, C), jnp.float32),
            jax.ShapeDtypeStruct((NP,), jnp.float32),
        ],
        mesh=mesh,
        compiler_params=pltpu.CompilerParams(use_tc_tiling_on_sc=False, needs_layout_passes=False),
        scratch_types=dict(
            acc=pltpu.VMEM_SHARED((NP, H), jnp.float32),
            deg_sh=pltpu.VMEM_SHARED((NP,), jnp.float32),
            dinv_sh=pltpu.VMEM_SHARED((NP,), jnp.float32),
            ewb=pltpu.VMEM((DB, C), jnp.float32),
            dstb2=pltpu.VMEM((DB, C), jnp.int32),
            degb=pltpu.VMEM((NPT,), jnp.float32),
            dslice=pltpu.VMEM((NPT,), jnp.float32),
            d2slice=pltpu.VMEM((NPT,), jnp.float32),
            dinvb=pltpu.VMEM((NP,), jnp.float32),
            srcb=pltpu.VMEM((AB, C), jnp.int32),
            dstb=pltpu.VMEM((AB, C), jnp.int32),
            ewbn=pltpu.VMEM((AB, C), jnp.float32),
            normb=pltpu.VMEM((AB, C), jnp.float32),
            rb0=pltpu.VMEM((C, H), jnp.float32),
            gsem=pltpu.SemaphoreType.DMA,
            ssem=pltpu.SemaphoreType.DMA,
        ),
    )
    def k(h1_hbm, src_hbm, dst_hbm, ew_hbm, part_out, norm_out, dinv2_out,
          acc, deg_sh, dinv_sh, ewb, dstb2, degb, dslice, d2slice, dinvb,
          srcb, dstb, ewbn, normb, rb0, gsem, ssem):
        cid = lax.axis_index("c")
        sid = lax.axis_index("s")
        wid = cid * NS + sid
        nbase = sid * NPT

        def zd(kk, carry):
            dinvb[pl.ds(kk * 16, 16)] = _zeros16f()
            return carry
        lax.fori_loop(0, NPT // 16, zd, 0)
        pltpu.sync_copy(dinvb.at[pl.ds(0, NPT)], deg_sh.at[pl.ds(nbase, NPT)])
        _zero_acc_slice(rb0, acc, nbase, NPT)
        plsc.subcore_barrier()

        ebase = sid * RPS
        def degblk(bi, carry):
            pltpu.sync_copy(ew_hbm.at[pl.ds(ebase + bi * DB, DB)], ewb)
            pltpu.sync_copy(dst_hbm.at[pl.ds(ebase + bi * DB, DB)], dstb2)
            def degbody(c, carry2):
                pltpu.sync_copy(ewb.at[c], deg_sh.at[dstb2.at[c]], add=True)
                return carry2
            lax.fori_loop(0, DB, degbody, 0)
            return carry
        lax.fori_loop(0, RPS // DB, degblk, 0)
        plsc.subcore_barrier()

        pltpu.sync_copy(deg_sh.at[pl.ds(nbase, NPT)], degb)
        def dbody(kk, carry):
            sl = pl.ds(kk * 16, 16)
            dv = degb[sl] + 1.0
            iv = jnp.int32(0x5F3759DF) - (plsc.bitcast(dv, jnp.int32) >> 1)
            y = plsc.bitcast(iv, jnp.float32)
            y = y * (1.5 - 0.5 * dv * y * y)
            y = y * (1.5 - 0.5 * dv * y * y)
            y = y * (1.5 - 0.5 * dv * y * y)
            dslice[sl] = y
            d2slice[sl] = y * y
            return carry
        lax.fori_loop(0, NPT // 16, dbody, 0)
        pltpu.sync_copy(dslice, dinv_sh.at[pl.ds(nbase, NPT)])

        @pl.when(cid == 0)
        def _():
            pltpu.sync_copy(d2slice, dinv2_out.at[pl.ds(nbase, NPT)])
        plsc.subcore_barrier()

        pltpu.sync_copy(dinv_sh, dinvb)
        abase = wid * RPT
        def aggblk(bi, carry):
            rbase = abase + bi * AB
            pltpu.sync_copy(src_hbm.at[pl.ds(rbase, AB)], srcb)
            pltpu.sync_copy(dst_hbm.at[pl.ds(rbase, AB)], dstb)
            pltpu.sync_copy(ew_hbm.at[pl.ds(rbase, AB)], ewbn)
            def chunk(c, carry2):
                gcp = pltpu.async_copy(h1_hbm.at[srcb.at[c]], rb0, gsem)
                for kk in range(C // 16):
                    sl = pl.ds(kk * 16, 16)
                    a = plsc.load_gather(dinvb, [srcb[c, sl]])
                    b = plsc.load_gather(dinvb, [dstb[c, sl]])
                    normb[c, sl] = a * ewbn[c, sl] * b
                gcp.wait()
                _scale_chunk(rb0, normb, c)
                pltpu.async_copy(rb0, acc.at[dstb.at[c]], ssem, add=True).wait()
                return carry2
            lax.fori_loop(0, AB, chunk, 0)
            pltpu.sync_copy(normb, norm_out.at[pl.ds(rbase, AB)])
            return carry
        lax.fori_loop(0, RPT // AB, aggblk, 0)
        plsc.subcore_barrier()

        pltpu.sync_copy(acc.at[pl.ds(nbase, NPT)],
                        part_out.at[cid, pl.ds(nbase, NPT)])

    return k(h1, src2d, dst2d, ew2d)


def _sc_layer2(h2, src2d, dst2d, norm2d, NP):
    ROWS = src2d.shape[0]
    RPT = ROWS // (NC * NS)
    NPT = NP // NS
    mesh = plsc.VectorSubcoreMesh(core_axis_name="c", subcore_axis_name="s")

    @functools.partial(
        pl.kernel,
        out_type=jax.ShapeDtypeStruct((NC, NP, H), jnp.float32),
        mesh=mesh,
        compiler_params=pltpu.CompilerParams(use_tc_tiling_on_sc=False, needs_layout_passes=False),
        scratch_types=dict(
            acc=pltpu.VMEM_SHARED((NP, H), jnp.float32),
            srcb=pltpu.VMEM((RPT, C), jnp.int32),
            dstb=pltpu.VMEM((RPT, C), jnp.int32),
            normb=pltpu.VMEM((RPT, C), jnp.float32),
            rb0=pltpu.VMEM((C, H), jnp.float32),
            gsem=pltpu.SemaphoreType.DMA,
            ssem=pltpu.SemaphoreType.DMA,
        ),
    )
    def k(h2_hbm, src_hbm, dst_hbm, norm_hbm, part_out,
          acc, srcb, dstb, normb, rb0, gsem, ssem):
        cid = lax.axis_index("c")
        sid = lax.axis_index("s")
        wid = cid * NS + sid
        nbase = sid * NPT
        _zero_acc_slice(rb0, acc, nbase, NPT)
        abase = wid * RPT
        pltpu.sync_copy(src_hbm.at[pl.ds(abase, RPT)], srcb)
        pltpu.sync_copy(dst_hbm.at[pl.ds(abase, RPT)], dstb)
        pltpu.sync_copy(norm_hbm.at[pl.ds(abase, RPT)], normb)
        plsc.subcore_barrier()
        _aggregate(h2_hbm, srcb, dstb, normb, acc, RPT, rb0, gsem, ssem)
        plsc.subcore_barrier()
        pltpu.sync_copy(acc.at[pl.ds(nbase, NPT)],
                        part_out.at[cid, pl.ds(nbase, NPT)])

    return k(h2, src2d, dst2d, norm2d)


_BR = 1280


def _tc_grid_call(body, n_out, NP, *args):
    specs = []
    for a in args:
        if a.ndim == 2 and a.shape[0] == NP:
            specs.append(pl.BlockSpec((_BR, a.shape[1]), lambda i: (i, 0)))
        else:
            specs.append(pl.BlockSpec(a.shape, lambda i, nd=a.ndim: (0,) * nd))
    outs = [jax.ShapeDtypeStruct((NP, H), jnp.float32)] * n_out
    return pl.pallas_call(
        body,
        grid=(NP // _BR,),
        in_specs=specs,
        out_specs=[pl.BlockSpec((_BR, H), lambda i: (i, 0))] * n_out,
        out_shape=outs,
    )(*args)


def _tc1_body(x, wgc1, wm1, bm1, wm2, bm2, h1o, hno):
    xb = x[...]
    h1o[...] = jnp.dot(xb, wgc1[...], preferred_element_type=jnp.float32)
    t = jnp.tanh(jnp.dot(xb, wm1[...], preferred_element_type=jnp.float32)
                 + bm1[...])
    hno[...] = (jnp.dot(t, wm2[...], preferred_element_type=jnp.float32)
                + bm2[...])


def _tc2_body(p0, p1, h1, d2, bgc1, wgc2, h2o):
    agg = p0[...] + p1[...] + d2[...] * h1[...] + bgc1[...]
    g = jnp.maximum(agg, 0.0)
    h2o[...] = jnp.dot(g, wgc2[...], preferred_element_type=jnp.float32)


def _tc3_body(p0, p1, h2, d2, bgc2, hn, wga, wgb, bg, dxo):
    agg = p0[...] + p1[...] + d2[...] * h2[...] + bgc2[...]
    hnb = hn[...]
    z = (jnp.dot(agg, wga[...], preferred_element_type=jnp.float32)
         + jnp.dot(hnb, wgb[...], preferred_element_type=jnp.float32)
         + bg[...])
    gate = jax.nn.sigmoid(z)
    dxo[...] = gate * agg + (1.0 - gate) * hnb


def kernel(t, x, edge_index, edge_weight, W_gc1, b_gc1, W_gc2, b_gc2,
           W_m1, b_m1, W_m2, b_m2, W_g, b_g):
    b_sz, n, h_dim = x.shape
    e_num = edge_weight.shape[0]
    assert h_dim == H and e_num % (C * NC * NS) == 0
    NP = ((n + NS * 16 - 1) // (NS * 16)) * (NS * 16)
    x_flat = x.reshape(n, h_dim)
    xp = jnp.pad(x_flat, ((0, NP - n), (0, 0)))
    src2d = edge_index[0].reshape(-1, C)
    dst2d = edge_index[1].reshape(-1, C)
    ew2d = edge_weight.reshape(-1, C)

    h1, hn = _tc_grid_call(_tc1_body, 2, NP, xp, W_gc1, W_m1,
                           b_m1.reshape(1, H), W_m2, b_m2.reshape(1, H))

    part1, norm2d, dinv2 = _sc_layer1(h1, src2d, dst2d, ew2d, NP)
    d2 = dinv2.reshape(NP, 1)

    (h2,) = _tc_grid_call(_tc2_body, 1, NP, part1[0], part1[1], h1, d2,
                          b_gc1.reshape(1, H), W_gc2)

    part2 = _sc_layer2(h2, src2d, dst2d, norm2d, NP)

    (dx,) = _tc_grid_call(_tc3_body, 1, NP, part2[0], part2[1], h2, d2,
                          b_gc2.reshape(1, H), hn, W_g[:H], W_g[H:],
                          b_g.reshape(1, H))
    return dx[:n].reshape(b_sz, n, h_dim)


# speedup vs baseline: 19.3550x; 1.4345x over previous
import functools

import jax
import jax.numpy as jnp
from jax import lax
from jax.experimental import pallas as pl
from jax.experimental.pallas import tpu as pltpu
from jax.experimental.pallas import tpu_sc as plsc

H = 128
C = 80
NC = 2
NS = 16
AB = 25
DB = 25

_SC_PARAMS = pltpu.CompilerParams(use_tc_tiling_on_sc=False,
                                  needs_layout_passes=False)


def _zeros16f():
    return jnp.zeros((16,), jnp.float32)


def _full16(v):
    return jnp.full((16,), v, jnp.int32)


def _scale_chunk(rb, normb, c):
    def body(e, carry):
        nv = plsc.load_gather(normb, [_full16(c), _full16(e)])
        for j in range(H // 16):
            sl = pl.ds(j * 16, 16)
            rb[e, sl] = rb[e, sl] * nv
        return carry
    lax.fori_loop(0, C, body, 0)


def _zero_acc_slice(rb0, acc, base, npt):
    def zbody(e, carry):
        for j in range(H // 16):
            rb0[e, pl.ds(j * 16, 16)] = _zeros16f()
        return carry
    lax.fori_loop(0, C, zbody, 0)
    for m in range(npt // C):
        pltpu.sync_copy(rb0, acc.at[pl.ds(base + m * C, C)])


def _agg_blocks(h_hbm, acc, srcb, dstb, normb, rb0, rb1, g0, g1, s0, s1,
                n_blocks, stage, finish):
    def wait_gather(rb, sem):
        pltpu.make_async_copy(h_hbm.at[pl.ds(0, C)], rb, sem).wait()

    def wait_scatter(rb, sem):
        pltpu.make_async_copy(rb, acc.at[pl.ds(0, C)], sem).wait()

    def block(bi, carry):
        stage(bi)
        pltpu.async_copy(h_hbm.at[srcb.at[0]], rb0, g0)
        pltpu.async_copy(h_hbm.at[srcb.at[1]], rb1, g1)

        def pair(p, carry2):
            c0 = 2 * p
            c1 = c0 + 1
            wait_gather(rb0, g0)
            _scale_chunk(rb0, normb, c0)
            pltpu.async_copy(rb0, acc.at[dstb.at[c0]], s0, add=True)
            wait_gather(rb1, g1)
            _scale_chunk(rb1, normb, c1)
            pltpu.async_copy(rb1, acc.at[dstb.at[c1]], s1, add=True)
            wait_scatter(rb0, s0)
            pltpu.async_copy(h_hbm.at[srcb.at[c0 + 2]], rb0, g0)
            wait_scatter(rb1, s1)

            @pl.when(p < (AB - 1) // 2 - 1)
            def _():
                pltpu.async_copy(h_hbm.at[srcb.at[c1 + 2]], rb1, g1)
            return carry2
        lax.fori_loop(0, (AB - 1) // 2, pair, 0)

        wait_gather(rb0, g0)
        _scale_chunk(rb0, normb, AB - 1)
        pltpu.async_copy(rb0, acc.at[dstb.at[AB - 1]], s0, add=True)
        wait_scatter(rb0, s0)
        finish(bi)
        return carry
    lax.fori_loop(0, n_blocks, block, 0)


def _sc_layer1(h1, src2d, dst2d, ew2d, NP):
    ROWS = src2d.shape[0]
    RPT = ROWS // (NC * NS)
    RPS = ROWS // NS
    NPT = NP // NS
    mesh = plsc.VectorSubcoreMesh(core_axis_name="c", subcore_axis_name="s")

    @functools.partial(
        pl.kernel,
        out_type=[
            jax.ShapeDtypeStruct((NC, NP, H), jnp.float32),
            jax.ShapeDtypeStruct((ROWS, C), jnp.float32),
            jax.ShapeDtypeStruct((NP,), jnp.float32),
        ],
        mesh=mesh,
        compiler_params=_SC_PARAMS,
        scratch_types=dict(
            acc=pltpu.VMEM_SHARED((NP, H), jnp.float32),
            deg_sh=pltpu.VMEM_SHARED((NP,), jnp.float32),
            dinv_sh=pltpu.VMEM_SHARED((NP,), jnp.float32),
            ewb=pltpu.VMEM((DB, C), jnp.float32),
            dstb2=pltpu.VMEM((DB, C), jnp.int32),
            degb=pltpu.VMEM((NPT,), jnp.float32),
            dslice=pltpu.VMEM((NPT,), jnp.float32),
            d2slice=pltpu.VMEM((NPT,), jnp.float32),
            dinvb=pltpu.VMEM((NP,), jnp.float32),
            srcb=pltpu.VMEM((AB, C), jnp.int32),
            dstb=pltpu.VMEM((AB, C), jnp.int32),
            ewbn=pltpu.VMEM((AB, C), jnp.float32),
            normb=pltpu.VMEM((AB, C), jnp.float32),
            rb0=pltpu.VMEM((C, H), jnp.float32),
            rb1=pltpu.VMEM((C, H), jnp.float32),
            g0=pltpu.SemaphoreType.DMA,
            g1=pltpu.SemaphoreType.DMA,
            s0=pltpu.SemaphoreType.DMA,
            s1=pltpu.SemaphoreType.DMA,
            dsem=pltpu.SemaphoreType.DMA,
        ),
    )
    def k(h1_hbm, src_hbm, dst_hbm, ew_hbm, part_out, norm_out, dinv2_out,
          acc, deg_sh, dinv_sh, ewb, dstb2, degb, dslice, d2slice, dinvb,
          srcb, dstb, ewbn, normb, rb0, rb1, g0, g1, s0, s1, dsem):
        cid = lax.axis_index("c")
        sid = lax.axis_index("s")
        wid = cid * NS + sid
        nbase = sid * NPT

        def zd(kk, carry):
            dinvb[pl.ds(kk * 16, 16)] = _zeros16f()
            return carry
        lax.fori_loop(0, NPT // 16, zd, 0)
        pltpu.sync_copy(dinvb.at[pl.ds(0, NPT)], deg_sh.at[pl.ds(nbase, NPT)])
        _zero_acc_slice(rb0, acc, nbase, NPT)
        plsc.subcore_barrier()

        ebase = sid * RPS
        def degblk(bi, carry):
            pltpu.sync_copy(ew_hbm.at[pl.ds(ebase + bi * DB, DB)], ewb)
            pltpu.sync_copy(dst_hbm.at[pl.ds(ebase + bi * DB, DB)], dstb2)
            def fire(c, carry2):
                pltpu.async_copy(ewb.at[c], deg_sh.at[dstb2.at[c]], dsem,
                                 add=True)
                return carry2
            lax.fori_loop(0, DB, fire, 0)
            def drain(c, carry2):
                pltpu.make_async_copy(ewb.at[0], deg_sh.at[pl.ds(0, C)],
                                      dsem).wait()
                return carry2
            lax.fori_loop(0, DB, drain, 0)
            return carry
        lax.fori_loop(0, RPS // DB, degblk, 0)
        plsc.subcore_barrier()

        pltpu.sync_copy(deg_sh.at[pl.ds(nbase, NPT)], degb)
        def dbody(kk, carry):
            sl = pl.ds(kk * 16, 16)
            dv = degb[sl] + 1.0
            iv = jnp.int32(0x5F3759DF) - (plsc.bitcast(dv, jnp.int32) >> 1)
            y = plsc.bitcast(iv, jnp.float32)
            y = y * (1.5 - 0.5 * dv * y * y)
            y = y * (1.5 - 0.5 * dv * y * y)
            y = y * (1.5 - 0.5 * dv * y * y)
            dslice[sl] = y
            d2slice[sl] = y * y
            return carry
        lax.fori_loop(0, NPT // 16, dbody, 0)
        pltpu.sync_copy(dslice, dinv_sh.at[pl.ds(nbase, NPT)])

        @pl.when(cid == 0)
        def _():
            pltpu.sync_copy(d2slice, dinv2_out.at[pl.ds(nbase, NPT)])
        plsc.subcore_barrier()

        pltpu.sync_copy(dinv_sh, dinvb)
        abase = wid * RPT

        def stage(bi):
            rbase = abase + bi * AB
            pltpu.sync_copy(src_hbm.at[pl.ds(rbase, AB)], srcb)
            pltpu.sync_copy(dst_hbm.at[pl.ds(rbase, AB)], dstb)
            pltpu.sync_copy(ew_hbm.at[pl.ds(rbase, AB)], ewbn)
            def nbody(c, carry):
                for kk in range(C // 16):
                    sl = pl.ds(kk * 16, 16)
                    a = plsc.load_gather(dinvb, [srcb[c, sl]])
                    b = plsc.load_gather(dinvb, [dstb[c, sl]])
                    normb[c, sl] = a * ewbn[c, sl] * b
                return carry
            lax.fori_loop(0, AB, nbody, 0)

        def finish(bi):
            pltpu.sync_copy(normb, norm_out.at[pl.ds(abase + bi * AB, AB)])

        _agg_blocks(h1_hbm, acc, srcb, dstb, normb, rb0, rb1, g0, g1, s0, s1,
                    RPT // AB, stage, finish)
        plsc.subcore_barrier()

        pltpu.sync_copy(acc.at[pl.ds(nbase, NPT)],
                        part_out.at[cid, pl.ds(nbase, NPT)])

    return k(h1, src2d, dst2d, ew2d)


def _sc_layer2(h2, src2d, dst2d, norm2d, NP):
    ROWS = src2d.shape[0]
    RPT = ROWS // (NC * NS)
    NPT = NP // NS
    mesh = plsc.VectorSubcoreMesh(core_axis_name="c", subcore_axis_name="s")

    @functools.partial(
        pl.kernel,
        out_type=jax.ShapeDtypeStruct((NC, NP, H), jnp.float32),
        mesh=mesh,
        compiler_params=_SC_PARAMS,
        scratch_types=dict(
            acc=pltpu.VMEM_SHARED((NP, H), jnp.float32),
            srcb=pltpu.VMEM((AB, C), jnp.int32),
            dstb=pltpu.VMEM((AB, C), jnp.int32),
            normb=pltpu.VMEM((AB, C), jnp.float32),
            rb0=pltpu.VMEM((C, H), jnp.float32),
            rb1=pltpu.VMEM((C, H), jnp.float32),
            g0=pltpu.SemaphoreType.DMA,
            g1=pltpu.SemaphoreType.DMA,
            s0=pltpu.SemaphoreType.DMA,
            s1=pltpu.SemaphoreType.DMA,
        ),
    )
    def k(h2_hbm, src_hbm, dst_hbm, norm_hbm, part_out,
          acc, srcb, dstb, normb, rb0, rb1, g0, g1, s0, s1):
        cid = lax.axis_index("c")
        sid = lax.axis_index("s")
        wid = cid * NS + sid
        nbase = sid * NPT
        _zero_acc_slice(rb0, acc, nbase, NPT)
        abase = wid * RPT
        plsc.subcore_barrier()

        def stage(bi):
            rbase = abase + bi * AB
            pltpu.sync_copy(src_hbm.at[pl.ds(rbase, AB)], srcb)
            pltpu.sync_copy(dst_hbm.at[pl.ds(rbase, AB)], dstb)
            pltpu.sync_copy(norm_hbm.at[pl.ds(rbase, AB)], normb)

        def finish(bi):
            pass

        _agg_blocks(h2_hbm, acc, srcb, dstb, normb, rb0, rb1, g0, g1, s0, s1,
                    RPT // AB, stage, finish)
        plsc.subcore_barrier()
        pltpu.sync_copy(acc.at[pl.ds(nbase, NPT)],
                        part_out.at[cid, pl.ds(nbase, NPT)])

    return k(h2, src2d, dst2d, norm2d)


_BR = 1280


def _tc_grid_call(body, n_out, NP, *args):
    specs = []
    for a in args:
        if a.ndim == 2 and a.shape[0] == NP:
            specs.append(pl.BlockSpec((_BR, a.shape[1]), lambda i: (i, 0)))
        else:
            specs.append(pl.BlockSpec(a.shape, lambda i, nd=a.ndim: (0,) * nd))
    outs = [jax.ShapeDtypeStruct((NP, H), jnp.float32)] * n_out
    return pl.pallas_call(
        body,
        grid=(NP // _BR,),
        in_specs=specs,
        out_specs=[pl.BlockSpec((_BR, H), lambda i: (i, 0))] * n_out,
        out_shape=outs,
    )(*args)


def _tc1_body(x, wgc1, wm1, bm1, wm2, bm2, h1o, hno):
    xb = x[...]
    h1o[...] = jnp.dot(xb, wgc1[...], preferred_element_type=jnp.float32)
    t = jnp.tanh(jnp.dot(xb, wm1[...], preferred_element_type=jnp.float32)
                 + bm1[...])
    hno[...] = (jnp.dot(t, wm2[...], preferred_element_type=jnp.float32)
                + bm2[...])


def _tc2_body(p0, p1, h1, d2, bgc1, wgc2, h2o):
    agg = p0[...] + p1[...] + d2[...] * h1[...] + bgc1[...]
    g = jnp.maximum(agg, 0.0)
    h2o[...] = jnp.dot(g, wgc2[...], preferred_element_type=jnp.float32)


def _tc3_body(p0, p1, h2, d2, bgc2, hn, wga, wgb, bg, dxo):
    agg = p0[...] + p1[...] + d2[...] * h2[...] + bgc2[...]
    hnb = hn[...]
    z = (jnp.dot(agg, wga[...], preferred_element_type=jnp.float32)
         + jnp.dot(hnb, wgb[...], preferred_element_type=jnp.float32)
         + bg[...])
    gate = jax.nn.sigmoid(z)
    dxo[...] = gate * agg + (1.0 - gate) * hnb


def kernel(t, x, edge_index, edge_weight, W_gc1, b_gc1, W_gc2, b_gc2,
           W_m1, b_m1, W_m2, b_m2, W_g, b_g):
    b_sz, n, h_dim = x.shape
    e_num = edge_weight.shape[0]
    assert h_dim == H and e_num % (C * NC * NS) == 0
    NP = ((n + NS * 16 - 1) // (NS * 16)) * (NS * 16)
    x_flat = x.reshape(n, h_dim)
    xp = jnp.pad(x_flat, ((0, NP - n), (0, 0)))
    src2d = edge_index[0].reshape(-1, C)
    dst2d = edge_index[1].reshape(-1, C)
    ew2d = edge_weight.reshape(-1, C)

    h1, hn = _tc_grid_call(_tc1_body, 2, NP, xp, W_gc1, W_m1,
                           b_m1.reshape(1, H), W_m2, b_m2.reshape(1, H))

    part1, norm2d, dinv2 = _sc_layer1(h1, src2d, dst2d, ew2d, NP)
    d2 = dinv2.reshape(NP, 1)

    (h2,) = _tc_grid_call(_tc2_body, 1, NP, part1[0], part1[1], h1, d2,
                          b_gc1.reshape(1, H), W_gc2)

    part2 = _sc_layer2(h2, src2d, dst2d, norm2d, NP)

    (dx,) = _tc_grid_call(_tc3_body, 1, NP, part2[0], part2[1], h2, d2,
                          b_gc2.reshape(1, H), hn, W_g[:H], W_g[H:],
                          b_g.reshape(1, H))
    return dx[:n].reshape(b_sz, n, h_dim)


# speedup vs baseline: 22.3548x; 1.1550x over previous
import functools

import jax
import jax.numpy as jnp
from jax import lax
from jax.experimental import pallas as pl
from jax.experimental.pallas import tpu as pltpu
from jax.experimental.pallas import tpu_sc as plsc

H = 128
C = 80
NC = 2
NS = 16
AB = 25
DB = 25

_SC_PARAMS = pltpu.CompilerParams(use_tc_tiling_on_sc=False,
                                  needs_layout_passes=False)


def _zeros16f():
    return jnp.zeros((16,), jnp.float32)


def _full16(v):
    return jnp.full((16,), v, jnp.int32)


def _scale_chunk(rb, normb, c):
    @plsc.parallel_loop(0, C, unroll=4)
    def body(e):
        nv = plsc.load_gather(normb, [_full16(c), _full16(e)])
        for j in range(H // 16):
            sl = pl.ds(j * 16, 16)
            rb[e, sl] = rb[e, sl] * nv


def _zero_acc_slice(rb0, acc, base, npt):
    def zbody(e, carry):
        for j in range(H // 16):
            rb0[e, pl.ds(j * 16, 16)] = _zeros16f()
        return carry
    lax.fori_loop(0, C, zbody, 0)
    for m in range(npt // C):
        pltpu.sync_copy(rb0, acc.at[pl.ds(base + m * C, C)])


def _agg_blocks(h_hbm, acc, srcb, dstb, normb, rb0, rb1, g0, g1, s0, s1,
                n_blocks, stage, finish):
    def wait_gather(rb, sem):
        pltpu.make_async_copy(h_hbm.at[pl.ds(0, C)], rb, sem).wait()

    def wait_scatter(rb, sem):
        pltpu.make_async_copy(rb, acc.at[pl.ds(0, C)], sem).wait()

    def block(bi, carry):
        stage(bi)
        pltpu.async_copy(h_hbm.at[srcb.at[0]], rb0, g0)
        pltpu.async_copy(h_hbm.at[srcb.at[1]], rb1, g1)

        def pair(p, carry2):
            c0 = 2 * p
            c1 = c0 + 1
            wait_gather(rb0, g0)
            _scale_chunk(rb0, normb, c0)
            pltpu.async_copy(rb0, acc.at[dstb.at[c0]], s0, add=True)
            wait_gather(rb1, g1)
            _scale_chunk(rb1, normb, c1)
            pltpu.async_copy(rb1, acc.at[dstb.at[c1]], s1, add=True)
            wait_scatter(rb0, s0)
            pltpu.async_copy(h_hbm.at[srcb.at[c0 + 2]], rb0, g0)
            wait_scatter(rb1, s1)

            @pl.when(p < (AB - 1) // 2 - 1)
            def _():
                pltpu.async_copy(h_hbm.at[srcb.at[c1 + 2]], rb1, g1)
            return carry2
        lax.fori_loop(0, (AB - 1) // 2, pair, 0)

        wait_gather(rb0, g0)
        _scale_chunk(rb0, normb, AB - 1)
        pltpu.async_copy(rb0, acc.at[dstb.at[AB - 1]], s0, add=True)
        wait_scatter(rb0, s0)
        finish(bi)
        return carry
    lax.fori_loop(0, n_blocks, block, 0)


def _sc_layer1(h1, src2d, dst2d, ew2d, NP):
    ROWS = src2d.shape[0]
    RPT = ROWS // (NC * NS)
    RPS = ROWS // NS
    NPT = NP // NS
    mesh = plsc.VectorSubcoreMesh(core_axis_name="c", subcore_axis_name="s")

    @functools.partial(
        pl.kernel,
        out_type=[
            jax.ShapeDtypeStruct((NC, NP, H), jnp.float32),
            jax.ShapeDtypeStruct((ROWS, C), jnp.float32),
            jax.ShapeDtypeStruct((NP,), jnp.float32),
        ],
        mesh=mesh,
        compiler_params=_SC_PARAMS,
        scratch_types=dict(
            acc=pltpu.VMEM_SHARED((NP, H), jnp.float32),
            deg_sh=pltpu.VMEM_SHARED((NP,), jnp.float32),
            dinv_sh=pltpu.VMEM_SHARED((NP,), jnp.float32),
            ewb=pltpu.VMEM((DB, C), jnp.float32),
            dstb2=pltpu.VMEM((DB, C), jnp.int32),
            degb=pltpu.VMEM((NPT,), jnp.float32),
            dslice=pltpu.VMEM((NPT,), jnp.float32),
            d2slice=pltpu.VMEM((NPT,), jnp.float32),
            dinvb=pltpu.VMEM((NP,), jnp.float32),
            srcb=pltpu.VMEM((AB, C), jnp.int32),
            dstb=pltpu.VMEM((AB, C), jnp.int32),
            ewbn=pltpu.VMEM((AB, C), jnp.float32),
            normb=pltpu.VMEM((AB, C), jnp.float32),
            rb0=pltpu.VMEM((C, H), jnp.float32),
            rb1=pltpu.VMEM((C, H), jnp.float32),
            g0=pltpu.SemaphoreType.DMA,
            g1=pltpu.SemaphoreType.DMA,
            s0=pltpu.SemaphoreType.DMA,
            s1=pltpu.SemaphoreType.DMA,
            dsem=pltpu.SemaphoreType.DMA,
        ),
    )
    def k(h1_hbm, src_hbm, dst_hbm, ew_hbm, part_out, norm_out, dinv2_out,
          acc, deg_sh, dinv_sh, ewb, dstb2, degb, dslice, d2slice, dinvb,
          srcb, dstb, ewbn, normb, rb0, rb1, g0, g1, s0, s1, dsem):
        cid = lax.axis_index("c")
        sid = lax.axis_index("s")
        wid = cid * NS + sid
        nbase = sid * NPT

        def zd(kk, carry):
            dinvb[pl.ds(kk * 16, 16)] = _zeros16f()
            return carry
        lax.fori_loop(0, NPT // 16, zd, 0)
        pltpu.sync_copy(dinvb.at[pl.ds(0, NPT)], deg_sh.at[pl.ds(nbase, NPT)])
        _zero_acc_slice(rb0, acc, nbase, NPT)
        plsc.subcore_barrier()

        ebase = sid * RPS
        def degblk(bi, carry):
            pltpu.sync_copy(ew_hbm.at[pl.ds(ebase + bi * DB, DB)], ewb)
            pltpu.sync_copy(dst_hbm.at[pl.ds(ebase + bi * DB, DB)], dstb2)
            def fire(c, carry2):
                pltpu.async_copy(ewb.at[c], deg_sh.at[dstb2.at[c]], dsem,
                                 add=True)
                return carry2
            lax.fori_loop(0, DB, fire, 0)
            def drain(c, carry2):
                pltpu.make_async_copy(ewb.at[0], deg_sh.at[pl.ds(0, C)],
                                      dsem).wait()
                return carry2
            lax.fori_loop(0, DB, drain, 0)
            return carry
        lax.fori_loop(0, RPS // DB, degblk, 0)
        plsc.subcore_barrier()

        pltpu.sync_copy(deg_sh.at[pl.ds(nbase, NPT)], degb)
        def dbody(kk, carry):
            sl = pl.ds(kk * 16, 16)
            dv = degb[sl] + 1.0
            iv = jnp.int32(0x5F3759DF) - (plsc.bitcast(dv, jnp.int32) >> 1)
            y = plsc.bitcast(iv, jnp.float32)
            y = y * (1.5 - 0.5 * dv * y * y)
            y = y * (1.5 - 0.5 * dv * y * y)
            y = y * (1.5 - 0.5 * dv * y * y)
            dslice[sl] = y
            d2slice[sl] = y * y
            return carry
        lax.fori_loop(0, NPT // 16, dbody, 0)
        pltpu.sync_copy(dslice, dinv_sh.at[pl.ds(nbase, NPT)])

        @pl.when(cid == 0)
        def _():
            pltpu.sync_copy(d2slice, dinv2_out.at[pl.ds(nbase, NPT)])
        plsc.subcore_barrier()

        pltpu.sync_copy(dinv_sh, dinvb)
        abase = wid * RPT

        def stage(bi):
            rbase = abase + bi * AB
            pltpu.sync_copy(src_hbm.at[pl.ds(rbase, AB)], srcb)
            pltpu.sync_copy(dst_hbm.at[pl.ds(rbase, AB)], dstb)
            pltpu.sync_copy(ew_hbm.at[pl.ds(rbase, AB)], ewbn)
            @plsc.parallel_loop(0, AB, unroll=2)
            def nbody(c):
                for kk in range(C // 16):
                    sl = pl.ds(kk * 16, 16)
                    a = plsc.load_gather(dinvb, [srcb[c, sl]])
                    b = plsc.load_gather(dinvb, [dstb[c, sl]])
                    normb[c, sl] = a * ewbn[c, sl] * b

        def finish(bi):
            pltpu.sync_copy(normb, norm_out.at[pl.ds(abase + bi * AB, AB)])

        _agg_blocks(h1_hbm, acc, srcb, dstb, normb, rb0, rb1, g0, g1, s0, s1,
                    RPT // AB, stage, finish)
        plsc.subcore_barrier()

        pltpu.sync_copy(acc.at[pl.ds(nbase, NPT)],
                        part_out.at[cid, pl.ds(nbase, NPT)])

    return k(h1, src2d, dst2d, ew2d)


def _sc_layer2(h2, src2d, dst2d, norm2d, NP):
    ROWS = src2d.shape[0]
    RPT = ROWS // (NC * NS)
    NPT = NP // NS
    mesh = plsc.VectorSubcoreMesh(core_axis_name="c", subcore_axis_name="s")

    @functools.partial(
        pl.kernel,
        out_type=jax.ShapeDtypeStruct((NC, NP, H), jnp.float32),
        mesh=mesh,
        compiler_params=_SC_PARAMS,
        scratch_types=dict(
            acc=pltpu.VMEM_SHARED((NP, H), jnp.float32),
            srcb=pltpu.VMEM((AB, C), jnp.int32),
            dstb=pltpu.VMEM((AB, C), jnp.int32),
            normb=pltpu.VMEM((AB, C), jnp.float32),
            rb0=pltpu.VMEM((C, H), jnp.float32),
            rb1=pltpu.VMEM((C, H), jnp.float32),
            g0=pltpu.SemaphoreType.DMA,
            g1=pltpu.SemaphoreType.DMA,
            s0=pltpu.SemaphoreType.DMA,
            s1=pltpu.SemaphoreType.DMA,
        ),
    )
    def k(h2_hbm, src_hbm, dst_hbm, norm_hbm, part_out,
          acc, srcb, dstb, normb, rb0, rb1, g0, g1, s0, s1):
        cid = lax.axis_index("c")
        sid = lax.axis_index("s")
        wid = cid * NS + sid
        nbase = sid * NPT
        _zero_acc_slice(rb0, acc, nbase, NPT)
        abase = wid * RPT
        plsc.subcore_barrier()

        def stage(bi):
            rbase = abase + bi * AB
            pltpu.sync_copy(src_hbm.at[pl.ds(rbase, AB)], srcb)
            pltpu.sync_copy(dst_hbm.at[pl.ds(rbase, AB)], dstb)
            pltpu.sync_copy(norm_hbm.at[pl.ds(rbase, AB)], normb)

        def finish(bi):
            pass

        _agg_blocks(h2_hbm, acc, srcb, dstb, normb, rb0, rb1, g0, g1, s0, s1,
                    RPT // AB, stage, finish)
        plsc.subcore_barrier()
        pltpu.sync_copy(acc.at[pl.ds(nbase, NPT)],
                        part_out.at[cid, pl.ds(nbase, NPT)])

    return k(h2, src2d, dst2d, norm2d)


_BR = 1280


def _tc_grid_call(body, n_out, NP, *args):
    specs = []
    for a in args:
        if a.ndim == 2 and a.shape[0] == NP:
            specs.append(pl.BlockSpec((_BR, a.shape[1]), lambda i: (i, 0)))
        else:
            specs.append(pl.BlockSpec(a.shape, lambda i, nd=a.ndim: (0,) * nd))
    outs = [jax.ShapeDtypeStruct((NP, H), jnp.float32)] * n_out
    return pl.pallas_call(
        body,
        grid=(NP // _BR,),
        in_specs=specs,
        out_specs=[pl.BlockSpec((_BR, H), lambda i: (i, 0))] * n_out,
        out_shape=outs,
    )(*args)


def _tc1_body(x, wgc1, wm1, bm1, wm2, bm2, h1o, hno):
    xb = x[...]
    h1o[...] = jnp.dot(xb, wgc1[...], preferred_element_type=jnp.float32)
    t = jnp.tanh(jnp.dot(xb, wm1[...], preferred_element_type=jnp.float32)
                 + bm1[...])
    hno[...] = (jnp.dot(t, wm2[...], preferred_element_type=jnp.float32)
                + bm2[...])


def _tc2_body(p0, p1, h1, d2, bgc1, wgc2, h2o):
    agg = p0[...] + p1[...] + d2[...] * h1[...] + bgc1[...]
    g = jnp.maximum(agg, 0.0)
    h2o[...] = jnp.dot(g, wgc2[...], preferred_element_type=jnp.float32)


def _tc3_body(p0, p1, h2, d2, bgc2, hn, wga, wgb, bg, dxo):
    agg = p0[...] + p1[...] + d2[...] * h2[...] + bgc2[...]
    hnb = hn[...]
    z = (jnp.dot(agg, wga[...], preferred_element_type=jnp.float32)
         + jnp.dot(hnb, wgb[...], preferred_element_type=jnp.float32)
         + bg[...])
    gate = jax.nn.sigmoid(z)
    dxo[...] = gate * agg + (1.0 - gate) * hnb


def kernel(t, x, edge_index, edge_weight, W_gc1, b_gc1, W_gc2, b_gc2,
           W_m1, b_m1, W_m2, b_m2, W_g, b_g):
    b_sz, n, h_dim = x.shape
    e_num = edge_weight.shape[0]
    assert h_dim == H and e_num % (C * NC * NS) == 0
    NP = ((n + NS * 16 - 1) // (NS * 16)) * (NS * 16)
    x_flat = x.reshape(n, h_dim)
    xp = jnp.pad(x_flat, ((0, NP - n), (0, 0)))
    src2d = edge_index[0].reshape(-1, C)
    dst2d = edge_index[1].reshape(-1, C)
    ew2d = edge_weight.reshape(-1, C)

    h1, hn = _tc_grid_call(_tc1_body, 2, NP, xp, W_gc1, W_m1,
                           b_m1.reshape(1, H), W_m2, b_m2.reshape(1, H))

    part1, norm2d, dinv2 = _sc_layer1(h1, src2d, dst2d, ew2d, NP)
    d2 = dinv2.reshape(NP, 1)

    (h2,) = _tc_grid_call(_tc2_body, 1, NP, part1[0], part1[1], h1, d2,
                          b_gc1.reshape(1, H), W_gc2)

    part2 = _sc_layer2(h2, src2d, dst2d, norm2d, NP)

    (dx,) = _tc_grid_call(_tc3_body, 1, NP, part2[0], part2[1], h2, d2,
                          b_gc2.reshape(1, H), hn, W_g[:H], W_g[H:],
                          b_g.reshape(1, H))
    return dx[:n].reshape(b_sz, n, h_dim)


# speedup vs baseline: 25.9276x; 1.1598x over previous
import functools

import jax
import jax.numpy as jnp
from jax import lax
from jax.experimental import pallas as pl
from jax.experimental.pallas import tpu as pltpu
from jax.experimental.pallas import tpu_sc as plsc

H = 128
C = 80
NC = 2
NS = 16
AB = 25

_SC_PARAMS = pltpu.CompilerParams(use_tc_tiling_on_sc=False,
                                  needs_layout_passes=False)


def _zeros16f():
    return jnp.zeros((16,), jnp.float32)


def _full16(v):
    return jnp.full((16,), v, jnp.int32)


def _scale_chunk(rb, normb, c):
    @plsc.parallel_loop(0, C, unroll=8)
    def body(e):
        nv = plsc.load_gather(normb, [_full16(c), _full16(e)])
        for j in range(H // 16):
            sl = pl.ds(j * 16, 16)
            rb[e, sl] = rb[e, sl] * nv


def _zero_acc_slice(rb0, acc, base, nrows):
    def zbody(e, carry):
        for j in range(H // 16):
            rb0[e, pl.ds(j * 16, 16)] = _zeros16f()
        return carry
    lax.fori_loop(0, C, zbody, 0)
    for m in range(nrows // C):
        pltpu.sync_copy(rb0, acc.at[pl.ds(base + m * C, C)])
    rem = nrows % C
    if rem:
        pltpu.sync_copy(rb0.at[pl.ds(0, rem)],
                        acc.at[pl.ds(base + (nrows // C) * C, rem)])


def _agg_blocks(h_hbm, acc, srcb, dstb, normb, rbs, gs, ss,
                n_blocks, stage, finish):
    def wait_gather(b):
        pltpu.make_async_copy(h_hbm.at[pl.ds(0, C)], rbs[b], gs[b]).wait()

    def wait_scatter(b):
        pltpu.make_async_copy(rbs[b], acc.at[pl.ds(0, C)], ss[b]).wait()

    def chunk(c, b, swait, gissue, gb):
        wait_gather(b)
        _scale_chunk(rbs[b], normb, c)
        pltpu.async_copy(rbs[b], acc.at[dstb.at[c]], ss[b], add=True)
        if swait is not None:
            wait_scatter(swait)
        if gissue is not None:
            pltpu.async_copy(h_hbm.at[srcb.at[gissue]], rbs[gb], gs[gb])

    def block(bi, carry):
        stage(bi)
        pltpu.async_copy(h_hbm.at[srcb.at[0]], rbs[0], gs[0])
        pltpu.async_copy(h_hbm.at[srcb.at[1]], rbs[1], gs[1])
        chunk(0, 0, None, 2, 2)

        def triple(tt, carry2):
            c = 3 * tt + 1
            chunk(c, 1, 0, c + 2, 0)
            chunk(c + 1, 2, 1, c + 3, 1)
            chunk(c + 2, 0, 2, c + 4, 2)
            return carry2
        lax.fori_loop(0, (AB - 4) // 3, triple, 0)

        chunk(AB - 3, 1, 0, AB - 1, 0)
        chunk(AB - 2, 2, 1, None, None)
        chunk(AB - 1, 0, 2, None, None)
        wait_scatter(0)
        finish(bi)
        return carry
    lax.fori_loop(0, n_blocks, block, 0)


def _sc_layer1(h1, src2d, dst2d, ew2d, n_nodes, NP):
    ROWS = src2d.shape[0]
    RPT = ROWS // (NC * NS)
    RPS = ROWS // NS
    NPT = NP // NS
    APT = n_nodes // NS
    mesh = plsc.VectorSubcoreMesh(core_axis_name="c", subcore_axis_name="s")

    @functools.partial(
        pl.kernel,
        out_type=[
            jax.ShapeDtypeStruct((NC, n_nodes, H), jnp.float32),
            jax.ShapeDtypeStruct((ROWS, C), jnp.float32),
            jax.ShapeDtypeStruct((NP,), jnp.float32),
        ],
        mesh=mesh,
        compiler_params=_SC_PARAMS,
        scratch_types=dict(
            acc=pltpu.VMEM_SHARED((n_nodes, H), jnp.float32),
            deg_sh=pltpu.VMEM_SHARED((NP,), jnp.float32),
            dinv_sh=pltpu.VMEM_SHARED((NP,), jnp.float32),
            degb=pltpu.VMEM((NPT,), jnp.float32),
            dinvb=pltpu.VMEM((NP,), jnp.float32),
            srcb=pltpu.VMEM((AB, C), jnp.int32),
            dstb=pltpu.VMEM((AB, C), jnp.int32),
            ewbn=pltpu.VMEM((AB, C), jnp.float32),
            rb0=pltpu.VMEM((C, H), jnp.float32),
            rb1=pltpu.VMEM((C, H), jnp.float32),
            rb2=pltpu.VMEM((C, H), jnp.float32),
            g0=pltpu.SemaphoreType.DMA,
            g1=pltpu.SemaphoreType.DMA,
            g2=pltpu.SemaphoreType.DMA,
            s0=pltpu.SemaphoreType.DMA,
            s1=pltpu.SemaphoreType.DMA,
            s2=pltpu.SemaphoreType.DMA,
            dsem=pltpu.SemaphoreType.DMA,
        ),
    )
    def k(h1_hbm, src_hbm, dst_hbm, ew_hbm, part_out, norm_out, dinv2_out,
          acc, deg_sh, dinv_sh, degb, dinvb, srcb, dstb, ewbn,
          rb0, rb1, rb2, g0, g1, g2, s0, s1, s2, dsem):
        cid = lax.axis_index("c")
        sid = lax.axis_index("s")
        wid = cid * NS + sid
        nbase = sid * NPT
        abase_n = sid * APT

        def zd(kk, carry):
            dinvb[pl.ds(kk * 16, 16)] = _zeros16f()
            return carry
        lax.fori_loop(0, NPT // 16, zd, 0)
        pltpu.sync_copy(dinvb.at[pl.ds(0, NPT)], deg_sh.at[pl.ds(nbase, NPT)])
        _zero_acc_slice(rb0, acc, abase_n, APT)
        plsc.subcore_barrier()

        ebase = sid * RPS
        def degblk(bi, carry):
            pltpu.sync_copy(ew_hbm.at[pl.ds(ebase + bi * AB, AB)], ewbn)
            pltpu.sync_copy(dst_hbm.at[pl.ds(ebase + bi * AB, AB)], dstb)
            def fire(c, carry2):
                pltpu.async_copy(ewbn.at[c], deg_sh.at[dstb.at[c]], dsem,
                                 add=True)
                return carry2
            lax.fori_loop(0, AB, fire, 0)
            def drain(c, carry2):
                pltpu.make_async_copy(ewbn.at[0], deg_sh.at[pl.ds(0, C)],
                                      dsem).wait()
                return carry2
            lax.fori_loop(0, AB, drain, 0)
            return carry
        lax.fori_loop(0, RPS // AB, degblk, 0)
        plsc.subcore_barrier()

        pltpu.sync_copy(deg_sh.at[pl.ds(nbase, NPT)], degb)
        def dbody(kk, carry):
            sl = pl.ds(kk * 16, 16)
            dv = degb[sl] + 1.0
            iv = jnp.int32(0x5F3759DF) - (plsc.bitcast(dv, jnp.int32) >> 1)
            y = plsc.bitcast(iv, jnp.float32)
            y = y * (1.5 - 0.5 * dv * y * y)
            y = y * (1.5 - 0.5 * dv * y * y)
            y = y * (1.5 - 0.5 * dv * y * y)
            dinvb[sl] = y
            degb[sl] = y * y
            return carry
        lax.fori_loop(0, NPT // 16, dbody, 0)
        pltpu.sync_copy(dinvb.at[pl.ds(0, NPT)], dinv_sh.at[pl.ds(nbase, NPT)])

        @pl.when(cid == 0)
        def _():
            pltpu.sync_copy(degb, dinv2_out.at[pl.ds(nbase, NPT)])
        plsc.subcore_barrier()

        pltpu.sync_copy(dinv_sh, dinvb)
        abase = wid * RPT

        def stage(bi):
            rbase = abase + bi * AB
            pltpu.sync_copy(src_hbm.at[pl.ds(rbase, AB)], srcb)
            pltpu.sync_copy(dst_hbm.at[pl.ds(rbase, AB)], dstb)
            pltpu.sync_copy(ew_hbm.at[pl.ds(rbase, AB)], ewbn)

            @plsc.parallel_loop(0, AB, unroll=2)
            def nbody(c):
                for kk in range(C // 16):
                    sl = pl.ds(kk * 16, 16)
                    a = plsc.load_gather(dinvb, [srcb[c, sl]])
                    b = plsc.load_gather(dinvb, [dstb[c, sl]])
                    ewbn[c, sl] = a * ewbn[c, sl] * b

        def finish(bi):
            pltpu.sync_copy(ewbn, norm_out.at[pl.ds(abase + bi * AB, AB)])

        _agg_blocks(h1_hbm, acc, srcb, dstb, ewbn, (rb0, rb1, rb2),
                    (g0, g1, g2), (s0, s1, s2), RPT // AB, stage, finish)
        plsc.subcore_barrier()

        pltpu.sync_copy(acc.at[pl.ds(abase_n, APT)],
                        part_out.at[cid, pl.ds(abase_n, APT)])

    return k(h1, src2d, dst2d, ew2d)


def _sc_layer2(h2, src2d, dst2d, norm2d, n_nodes):
    ROWS = src2d.shape[0]
    RPT = ROWS // (NC * NS)
    APT = n_nodes // NS
    mesh = plsc.VectorSubcoreMesh(core_axis_name="c", subcore_axis_name="s")

    @functools.partial(
        pl.kernel,
        out_type=jax.ShapeDtypeStruct((NC, n_nodes, H), jnp.float32),
        mesh=mesh,
        compiler_params=_SC_PARAMS,
        scratch_types=dict(
            acc=pltpu.VMEM_SHARED((n_nodes, H), jnp.float32),
            srcb=pltpu.VMEM((AB, C), jnp.int32),
            dstb=pltpu.VMEM((AB, C), jnp.int32),
            normb=pltpu.VMEM((AB, C), jnp.float32),
            rb0=pltpu.VMEM((C, H), jnp.float32),
            rb1=pltpu.VMEM((C, H), jnp.float32),
            rb2=pltpu.VMEM((C, H), jnp.float32),
            g0=pltpu.SemaphoreType.DMA,
            g1=pltpu.SemaphoreType.DMA,
            g2=pltpu.SemaphoreType.DMA,
            s0=pltpu.SemaphoreType.DMA,
            s1=pltpu.SemaphoreType.DMA,
            s2=pltpu.SemaphoreType.DMA,
        ),
    )
    def k(h2_hbm, src_hbm, dst_hbm, norm_hbm, part_out,
          acc, srcb, dstb, normb, rb0, rb1, rb2, g0, g1, g2, s0, s1, s2):
        cid = lax.axis_index("c")
        sid = lax.axis_index("s")
        wid = cid * NS + sid
        abase_n = sid * APT
        _zero_acc_slice(rb0, acc, abase_n, APT)
        abase = wid * RPT
        plsc.subcore_barrier()

        def stage(bi):
            rbase = abase + bi * AB
            pltpu.sync_copy(src_hbm.at[pl.ds(rbase, AB)], srcb)
            pltpu.sync_copy(dst_hbm.at[pl.ds(rbase, AB)], dstb)
            pltpu.sync_copy(norm_hbm.at[pl.ds(rbase, AB)], normb)

        def finish(bi):
            pass

        _agg_blocks(h2_hbm, acc, srcb, dstb, normb, (rb0, rb1, rb2),
                    (g0, g1, g2), (s0, s1, s2), RPT // AB, stage, finish)
        plsc.subcore_barrier()
        pltpu.sync_copy(acc.at[pl.ds(abase_n, APT)],
                        part_out.at[cid, pl.ds(abase_n, APT)])

    return k(h2, src2d, dst2d, norm2d)


_BR = 1000


def _tc_grid_call(body, n_out, n_nodes, *args):
    specs = []
    for a in args:
        if a.ndim == 2 and a.shape[0] == n_nodes:
            specs.append(pl.BlockSpec((_BR, a.shape[1]), lambda i: (i, 0)))
        else:
            specs.append(pl.BlockSpec(a.shape, lambda i, nd=a.ndim: (0,) * nd))
    outs = [jax.ShapeDtypeStruct((n_nodes, H), jnp.float32)] * n_out
    return pl.pallas_call(
        body,
        grid=(n_nodes // _BR,),
        in_specs=specs,
        out_specs=[pl.BlockSpec((_BR, H), lambda i: (i, 0))] * n_out,
        out_shape=outs,
    )(*args)


def _tc1_body(x, wgc1, wm1, bm1, wm2, bm2, h1o, hno):
    xb = x[...]
    h1o[...] = jnp.dot(xb, wgc1[...], preferred_element_type=jnp.float32)
    t = jnp.tanh(jnp.dot(xb, wm1[...], preferred_element_type=jnp.float32)
                 + bm1[...])
    hno[...] = (jnp.dot(t, wm2[...], preferred_element_type=jnp.float32)
                + bm2[...])


def _tc2_body(p0, p1, h1, d2, bgc1, wgc2, h2o):
    agg = p0[...] + p1[...] + d2[...] * h1[...] + bgc1[...]
    g = jnp.maximum(agg, 0.0)
    h2o[...] = jnp.dot(g, wgc2[...], preferred_element_type=jnp.float32)


def _tc3_body(p0, p1, h2, d2, bgc2, hn, wga, wgb, bg, dxo):
    agg = p0[...] + p1[...] + d2[...] * h2[...] + bgc2[...]
    hnb = hn[...]
    z = (jnp.dot(agg, wga[...], preferred_element_type=jnp.float32)
         + jnp.dot(hnb, wgb[...], preferred_element_type=jnp.float32)
         + bg[...])
    gate = jax.nn.sigmoid(z)
    dxo[...] = gate * agg + (1.0 - gate) * hnb


def kernel(t, x, edge_index, edge_weight, W_gc1, b_gc1, W_gc2, b_gc2,
           W_m1, b_m1, W_m2, b_m2, W_g, b_g):
    b_sz, n, h_dim = x.shape
    e_num = edge_weight.shape[0]
    assert h_dim == H and e_num % (C * NC * NS * AB) == 0
    assert n % NS == 0 and n % _BR == 0
    NP = ((n + NS * 16 - 1) // (NS * 16)) * (NS * 16)
    x_flat = x.reshape(n, h_dim)
    src2d = edge_index[0].reshape(-1, C)
    dst2d = edge_index[1].reshape(-1, C)
    ew2d = edge_weight.reshape(-1, C)

    h1, hn = _tc_grid_call(_tc1_body, 2, n, x_flat, W_gc1, W_m1,
                           b_m1.reshape(1, H), W_m2, b_m2.reshape(1, H))

    part1, norm2d, dinv2 = _sc_layer1(h1, src2d, dst2d, ew2d, n, NP)
    d2 = dinv2[:n].reshape(n, 1)

    (h2,) = _tc_grid_call(_tc2_body, 1, n, part1[0], part1[1], h1, d2,
                          b_gc1.reshape(1, H), W_gc2)

    part2 = _sc_layer2(h2, src2d, dst2d, norm2d, n)

    (dx,) = _tc_grid_call(_tc3_body, 1, n, part2[0], part2[1], h2, d2,
                          b_gc2.reshape(1, H), hn, W_g[:H], W_g[H:],
                          b_g.reshape(1, H))
    return dx.reshape(b_sz, n, h_dim)


# speedup vs baseline: 27.6096x; 1.0649x over previous
import functools

import jax
import jax.numpy as jnp
from jax import lax
from jax.experimental import pallas as pl
from jax.experimental.pallas import tpu as pltpu
from jax.experimental.pallas import tpu_sc as plsc

H = 128
HW = H // 2
C = 80
NC = 2
NS = 16
AB = 25

_SC_PARAMS = pltpu.CompilerParams(use_tc_tiling_on_sc=False,
                                  needs_layout_passes=False)


def _zeros16f():
    return jnp.zeros((16,), jnp.float32)


def _full16(v):
    return jnp.full((16,), v, jnp.int32)


def _scale_chunk(rbb, rf, wb, c):
    @plsc.parallel_loop(0, C, unroll=8)
    def body(e):
        nv = plsc.load_gather(wb, [_full16(c), _full16(e)])
        for j in range(HW // 16):
            w = rbb[e, pl.ds(j * 16, 16)]
            lo = plsc.bitcast(w << 16, jnp.float32) * nv
            hi = plsc.bitcast(w & jnp.int32(-65536), jnp.float32) * nv
            rf[e, pl.ds(32 * j, 16)] = lo
            rf[e, pl.ds(32 * j + 16, 16)] = hi


def _zero_acc_slice(rf0, acc, base, nrows):
    def zbody(e, carry):
        for j in range(H // 16):
            rf0[e, pl.ds(j * 16, 16)] = _zeros16f()
        return carry
    lax.fori_loop(0, C, zbody, 0)
    for m in range(nrows // C):
        pltpu.sync_copy(rf0, acc.at[pl.ds(base + m * C, C)])
    rem = nrows % C
    if rem:
        pltpu.sync_copy(rf0.at[pl.ds(0, rem)],
                        acc.at[pl.ds(base + (nrows // C) * C, rem)])


def _sc_deg(dst2d, ew2d, NP):
    ROWS = dst2d.shape[0]
    RPT = ROWS // (NC * NS)
    NPT = NP // NS
    mesh = plsc.VectorSubcoreMesh(core_axis_name="c", subcore_axis_name="s")

    @functools.partial(
        pl.kernel,
        out_type=jax.ShapeDtypeStruct((NC, NP), jnp.float32),
        mesh=mesh,
        compiler_params=_SC_PARAMS,
        scratch_types=dict(
            deg_sh=pltpu.VMEM_SHARED((NP,), jnp.float32),
            vb=pltpu.VMEM((AB, C), jnp.float32),
            ib=pltpu.VMEM((AB, C), jnp.int32),
            zb=pltpu.VMEM((NPT,), jnp.float32),
            dsem=pltpu.SemaphoreType.DMA,
        ),
    )
    def k(dst_hbm, ew_hbm, deg_out, deg_sh, vb, ib, zb, dsem):
        cid = lax.axis_index("c")
        sid = lax.axis_index("s")
        nbase = sid * NPT

        def zd(kk, carry):
            zb[pl.ds(kk * 16, 16)] = _zeros16f()
            return carry
        lax.fori_loop(0, NPT // 16, zd, 0)
        pltpu.sync_copy(zb, deg_sh.at[pl.ds(nbase, NPT)])
        plsc.subcore_barrier()

        ebase = (cid * NS + sid) * RPT
        def degblk(bi, carry):
            pltpu.sync_copy(ew_hbm.at[pl.ds(ebase + bi * AB, AB)], vb)
            pltpu.sync_copy(dst_hbm.at[pl.ds(ebase + bi * AB, AB)], ib)
            def fire(c, carry2):
                pltpu.async_copy(vb.at[c], deg_sh.at[ib.at[c]], dsem,
                                 add=True)
                return carry2
            lax.fori_loop(0, AB, fire, 0)
            def drain(c, carry2):
                pltpu.make_async_copy(vb.at[0], deg_sh.at[pl.ds(0, C)],
                                      dsem).wait()
                return carry2
            lax.fori_loop(0, AB, drain, 0)
            return carry
        lax.fori_loop(0, RPT // AB, degblk, 0)
        plsc.subcore_barrier()

        pltpu.sync_copy(deg_sh.at[pl.ds(nbase, NPT)],
                        deg_out.at[cid, pl.ds(nbase, NPT)])

    return k(dst2d, ew2d)


def _sc_agg(h_i32, src2d, dst2d, ew2d, n_nodes):
    ROWS = src2d.shape[0]
    RPT = ROWS // (NC * NS)
    APT = n_nodes // NS
    mesh = plsc.VectorSubcoreMesh(core_axis_name="c", subcore_axis_name="s")

    @functools.partial(
        pl.kernel,
        out_type=jax.ShapeDtypeStruct((NC, n_nodes, H), jnp.float32),
        mesh=mesh,
        compiler_params=_SC_PARAMS,
        scratch_types=dict(
            acc=pltpu.VMEM_SHARED((n_nodes, H), jnp.float32),
            srcb=pltpu.VMEM((AB, C), jnp.int32),
            dstb=pltpu.VMEM((AB, C), jnp.int32),
            wb=pltpu.VMEM((AB, C), jnp.float32),
            rbb0=pltpu.VMEM((C, HW), jnp.int32),
            rbb1=pltpu.VMEM((C, HW), jnp.int32),
            rbb2=pltpu.VMEM((C, HW), jnp.int32),
            rbb3=pltpu.VMEM((C, HW), jnp.int32),
            rf0=pltpu.VMEM((C, H), jnp.float32),
            rf1=pltpu.VMEM((C, H), jnp.float32),
            g0=pltpu.SemaphoreType.DMA,
            g1=pltpu.SemaphoreType.DMA,
            g2=pltpu.SemaphoreType.DMA,
            g3=pltpu.SemaphoreType.DMA,
            s0=pltpu.SemaphoreType.DMA,
            s1=pltpu.SemaphoreType.DMA,
        ),
    )
    def k(h_hbm, src_hbm, dst_hbm, ew_hbm, part_out,
          acc, srcb, dstb, wb, rbb0, rbb1, rbb2, rbb3, rf0, rf1,
          g0, g1, g2, g3, s0, s1):
        cid = lax.axis_index("c")
        sid = lax.axis_index("s")
        wid = cid * NS + sid
        nbase = sid * APT
        rbbs = (rbb0, rbb1, rbb2, rbb3)
        rfs = (rf0, rf1)
        gs = (g0, g1, g2, g3)
        ss = (s0, s1)

        _zero_acc_slice(rf0, acc, nbase, APT)
        plsc.subcore_barrier()

        abase = wid * RPT

        def wait_gather(b):
            pltpu.make_async_copy(h_hbm.at[pl.ds(0, C)], rbbs[b],
                                  gs[b]).wait()

        def wait_scatter(b):
            pltpu.make_async_copy(rfs[b], acc.at[pl.ds(0, C)],
                                  ss[b]).wait()

        def chunk(c, b4, b2, swait, gissue):
            wait_gather(b4)
            if swait:
                wait_scatter(b2)
            _scale_chunk(rbbs[b4], rfs[b2], wb, c)
            pltpu.async_copy(rfs[b2], acc.at[dstb.at[c]], ss[b2], add=True)
            if gissue is not None:
                gb = gissue % 4 if isinstance(gissue, int) else None
                pltpu.async_copy(h_hbm.at[srcb.at[gissue]], rbbs[gb],
                                 gs[gb])

        def chunk_t(c, b4, b2, gb):
            wait_gather(b4)
            wait_scatter(b2)
            _scale_chunk(rbbs[b4], rfs[b2], wb, c)
            pltpu.async_copy(rfs[b2], acc.at[dstb.at[c]], ss[b2], add=True)
            pltpu.async_copy(h_hbm.at[srcb.at[c + 3]], rbbs[gb], gs[gb])

        def block(bi, carry):
            rbase = abase + bi * AB
            pltpu.sync_copy(src_hbm.at[pl.ds(rbase, AB)], srcb)
            pltpu.sync_copy(dst_hbm.at[pl.ds(rbase, AB)], dstb)
            pltpu.sync_copy(ew_hbm.at[pl.ds(rbase, AB)], wb)

            pltpu.async_copy(h_hbm.at[srcb.at[0]], rbb0, g0)
            pltpu.async_copy(h_hbm.at[srcb.at[1]], rbb1, g1)
            pltpu.async_copy(h_hbm.at[srcb.at[2]], rbb2, g2)
            chunk(0, 0, 0, False, 3)
            chunk(1, 1, 1, False, 4)

            def quad(tt, carry2):
                c = 4 * tt + 2
                chunk_t(c, 2, 0, 1)
                chunk_t(c + 1, 3, 1, 2)
                chunk_t(c + 2, 0, 0, 3)
                chunk_t(c + 3, 1, 1, 0)
                return carry2
            lax.fori_loop(0, (AB - 5) // 4, quad, 0)

            chunk(22, 2, 0, True, None)
            chunk(23, 3, 1, True, None)
            chunk(24, 0, 0, True, None)
            wait_scatter(1)
            wait_scatter(0)
            return carry
        lax.fori_loop(0, RPT // AB, block, 0)
        plsc.subcore_barrier()

        pltpu.sync_copy(acc.at[pl.ds(nbase, APT)],
                        part_out.at[cid, pl.ds(nbase, APT)])

    return k(h_i32, src2d, dst2d, ew2d)


_BR = 1000


def _row_spec(cols):
    return pl.BlockSpec((_BR, cols), lambda i: (i, 0))


def _full_spec(a):
    return pl.BlockSpec(a.shape, lambda i, nd=a.ndim: (0,) * nd)


def _tc1_body(x, dp0, dp1, wgc1, wm1, bm1, wm2, bm2, h1po, hno, dvo):
    xb = x[...]
    dv = lax.rsqrt(dp0[...] + dp1[...] + 1.0)
    dvo[...] = dv
    h1po[...] = dv * jnp.dot(xb, wgc1[...],
                             preferred_element_type=jnp.float32)
    tnh = jnp.tanh(jnp.dot(xb, wm1[...], preferred_element_type=jnp.float32)
                   + bm1[...])
    hno[...] = (jnp.dot(tnh, wm2[...], preferred_element_type=jnp.float32)
                + bm2[...])


def _tc2_body(p0, p1, h1p, dv, bgc1, wgc2, h2po):
    agg = dv[...] * (p0[...] + p1[...] + h1p[...]) + bgc1[...]
    g = jnp.maximum(agg, 0.0)
    h2po[...] = dv[...] * jnp.dot(g, wgc2[...],
                                  preferred_element_type=jnp.float32)


def _tc3_body(p0, p1, h2p, dv, bgc2, hn, wga, wgb, bg, dxo):
    agg = dv[...] * (p0[...] + p1[...] + h2p[...]) + bgc2[...]
    hnb = hn[...]
    z = (jnp.dot(agg, wga[...], preferred_element_type=jnp.float32)
         + jnp.dot(hnb, wgb[...], preferred_element_type=jnp.float32)
         + bg[...])
    gate = jax.nn.sigmoid(z)
    dxo[...] = gate * agg + (1.0 - gate) * hnb


def _pack_bf(hp, n):
    shuf = jnp.transpose(hp.reshape(n, 4, 2, 16), (0, 1, 3, 2))
    return lax.bitcast_convert_type(
        shuf.astype(jnp.bfloat16).reshape(n, HW, 2), jnp.int32)


def kernel(t, x, edge_index, edge_weight, W_gc1, b_gc1, W_gc2, b_gc2,
           W_m1, b_m1, W_m2, b_m2, W_g, b_g):
    b_sz, n, h_dim = x.shape
    e_num = edge_weight.shape[0]
    assert h_dim == H and e_num % (C * NC * NS * AB) == 0
    assert n % NS == 0 and n % _BR == 0
    NP = ((n + NS * 16 - 1) // (NS * 16)) * (NS * 16)
    x_flat = x.reshape(n, h_dim)
    src2d = edge_index[0].reshape(-1, C)
    dst2d = edge_index[1].reshape(-1, C)
    ew2d = edge_weight.reshape(-1, C)
    grid = (n // _BR,)

    deg_part = _sc_deg(dst2d, ew2d, NP)
    dp0 = deg_part[0, :n].reshape(n, 1)
    dp1 = deg_part[1, :n].reshape(n, 1)

    h1p, hn, dv = pl.pallas_call(
        _tc1_body,
        grid=grid,
        in_specs=[_row_spec(H), _row_spec(1), _row_spec(1),
                  _full_spec(W_gc1), _full_spec(W_m1),
                  pl.BlockSpec((1, H), lambda i: (0, 0)),
                  _full_spec(W_m2), pl.BlockSpec((1, H), lambda i: (0, 0))],
        out_specs=[_row_spec(H), _row_spec(H), _row_spec(1)],
        out_shape=[jax.ShapeDtypeStruct((n, H), jnp.float32),
                   jax.ShapeDtypeStruct((n, H), jnp.float32),
                   jax.ShapeDtypeStruct((n, 1), jnp.float32)],
    )(x_flat, dp0, dp1, W_gc1, W_m1, b_m1.reshape(1, H), W_m2,
      b_m2.reshape(1, H))

    part1 = _sc_agg(_pack_bf(h1p, n), src2d, dst2d, ew2d, n)

    (h2p,) = pl.pallas_call(
        _tc2_body,
        grid=grid,
        in_specs=[_row_spec(H), _row_spec(H), _row_spec(H), _row_spec(1),
                  pl.BlockSpec((1, H), lambda i: (0, 0)), _full_spec(W_gc2)],
        out_specs=[_row_spec(H)],
        out_shape=[jax.ShapeDtypeStruct((n, H), jnp.float32)],
    )(part1[0], part1[1], h1p, dv, b_gc1.reshape(1, H), W_gc2)

    part2 = _sc_agg(_pack_bf(h2p, n), src2d, dst2d, ew2d, n)

    (dx,) = pl.pallas_call(
        _tc3_body,
        grid=grid,
        in_specs=[_row_spec(H), _row_spec(H), _row_spec(H), _row_spec(1),
                  pl.BlockSpec((1, H), lambda i: (0, 0)), _row_spec(H),
                  _full_spec(W_g[:H]), _full_spec(W_g[H:]),
                  pl.BlockSpec((1, H), lambda i: (0, 0))],
        out_specs=[_row_spec(H)],
        out_shape=[jax.ShapeDtypeStruct((n, H), jnp.float32)],
    )(part2[0], part2[1], h2p, dv, b_gc2.reshape(1, H), hn,
      W_g[:H], W_g[H:], b_g.reshape(1, H))
    return dx.reshape(b_sz, n, h_dim)
